# baseline (speedup 1.0000x reference)
import functools

import jax
import jax.numpy as jnp
from jax import lax
from jax.experimental import pallas as pl
from jax.experimental.pallas import tpu as pltpu
from jax.experimental.pallas import tpu_sc as plsc

B = 1024
T = 200
E = 64
N = B * T
V = 1000000

NC = 2
NS = 16
L = 16
NW = NC * NS

TK = 128
NTCH = V // TK
J1 = ((NTCH + NW - 1) // NW + 1) // 2 * 2
TAIL0 = NTCH * TK

PER_W = N // NW
CH = 320
NCHUNK = PER_W // CH

_MESH = dict(core_axis_name="c", subcore_axis_name="s",
             num_cores=NC, num_subcores=NS)


def _sc_transpose(tblT):
    @functools.partial(
        pl.kernel,
        out_type=jax.ShapeDtypeStruct((V // 2, 2 * E), jnp.float32),
        mesh=plsc.VectorSubcoreMesh(**_MESH),
        compiler_params=pltpu.CompilerParams(needs_layout_passes=False),
        scratch_types=[
            pltpu.VMEM((E, TK), jnp.float32),
            pltpu.VMEM((E, TK), jnp.float32),
            pltpu.VMEM((TK // 2, 2 * E), jnp.float32),
            pltpu.VMEM((TK // 2, 2 * E), jnp.float32),
        ],
    )
    def k1(tblT_hbm, tblR_hbm, sT0, sT1, oT0, oT1):
        sT_b = (sT0, sT1)
        oT_b = (oT0, oT1)
        wid = lax.axis_index("c") * NS + lax.axis_index("s")
        ri = [lax.iota(jnp.int32, L) + c * L for c in range(E // L)]

        def chunk_of(j):
            g = wid + j * NW
            return jnp.where(g < NTCH, g, NTCH - 1)

        def stage(j, b):
            tok0 = pl.multiple_of(chunk_of(j) * TK, 128)
            pltpu.sync_copy(tblT_hbm.at[:, pl.ds(tok0, TK)], sT_b[b])

        def consume(j, b):
            sT = sT_b[b]
            oT = oT_b[b]

            def t_body(t, _):
                ci = jnp.full((L,), t, jnp.int32)
                pr = lax.shift_right_logical(t, 1)
                ho = (t & 1) * E
                for c in range(E // L):
                    oT[pr, pl.ds(ho + c * L, L)] = plsc.load_gather(
                        sT, [ri[c], ci]
                    )
                return 0

            lax.fori_loop(0, TK, t_body, 0)
            row0 = pl.multiple_of(chunk_of(j) * (TK // 2), 8)
            pltpu.sync_copy(oT, tblR_hbm.at[pl.ds(row0, TK // 2)])

        stage(0, 0)

        def pair_body(p, _):
            j0 = 2 * p
            stage(j0 + 1, 1)
            consume(j0, 0)
            stage(j0 + 2, 0)
            consume(j0 + 1, 1)
            return 0

        lax.fori_loop(0, J1 // 2 - 1, pair_body, 0)
        stage(J1 - 1, 1)
        consume(J1 - 2, 0)
        consume(J1 - 1, 1)

    return k1(tblT)


def _sc_gather(xf, tblR, pos2, tail):
    @functools.partial(
        pl.kernel,
        out_type=jax.ShapeDtypeStruct((N // 2, 2 * E), jnp.float32),
        mesh=plsc.VectorSubcoreMesh(**_MESH),
        scratch_types=[
            pltpu.VMEM((CH,), jnp.int32),
            pltpu.VMEM((CH,), jnp.int32),
            pltpu.VMEM((CH,), jnp.int32),
            pltpu.VMEM((CH,), jnp.int32),
            pltpu.VMEM((CH, 2 * E), jnp.float32),
            pltpu.VMEM((CH, 2 * E), jnp.float32),
            pltpu.VMEM((CH // 2, 2 * E), jnp.float32),
            pltpu.VMEM((T // 2, 2 * E), jnp.float32),
            pltpu.VMEM(((V - TAIL0) // 2, 2 * E), jnp.float32),
            pltpu.SemaphoreType.DMA,
            pltpu.SemaphoreType.DMA,
        ],
    )
    def k2(x_hbm, tbl_hbm, pos_hbm, tail_hbm, out_hbm,
           idx_v0, idx_v1, pidx_v0, pidx_v1, rows_v0, rows_v1,
           out_v, pos_v, tail_v, gsem0, gsem1):
        idx_b = (idx_v0, idx_v1)
        pidx_b = (pidx_v0, pidx_v1)
        rows_b = (rows_v0, rows_v1)
        gsem_b = (gsem0, gsem1)
        wid = lax.axis_index("c") * NS + lax.axis_index("s")
        base = wid * PER_W
        pltpu.sync_copy(pos_hbm, pos_v)
        pltpu.sync_copy(tail_hbm, tail_v)

        def stage(i, b):
            off = pl.multiple_of(base + i * CH, 8)
            idx_v = idx_b[b]
            pidx_v = pidx_b[b]
            pltpu.sync_copy(x_hbm.at[pl.ds(off, CH)], idx_v)

            def shift_body(v, _):
                pidx_v[pl.ds(v * L, L)] = jnp.minimum(
                    lax.shift_right_logical(idx_v[pl.ds(v * L, L)], 1),
                    TAIL0 // 2 - 1,
                )
                return 0

            lax.fori_loop(0, CH // L, shift_body, 0)
            pltpu.async_copy(tbl_hbm.at[pidx_v], rows_b[b], gsem_b[b])

        def consume(i, b):
            idx_v = idx_b[b]
            rows_v = rows_b[b]
            pltpu.make_async_copy(
                tbl_hbm.at[pidx_b[b]], rows_v, gsem_b[b]
            ).wait()
            t0 = lax.rem(i * CH, T)

            def grp_body(m, _):
                iv = idx_v[pl.ds(m * L, L)]
                for r in range(L):
                    tok = m * L + r
                    a = (iv[r] & 1) * E
                    istail = iv[r] >= TAIL0
                    toff = jnp.maximum(iv[r] - TAIL0, 0)
                    tr = lax.shift_right_logical(toff, 1)
                    th = (toff & 1) * E
                    tt = t0 + tok
                    tt = jnp.where(tt >= T, tt - T, tt)
                    tt = jnp.where(tt >= T, tt - T, tt)
                    pr2 = lax.shift_right_logical(tt, 1)
                    ph = (tt & 1) * E
                    orow = lax.shift_right_logical(tok, 1)
                    oh = (tok & 1) * E
                    for c in range(E // L):
                        val = rows_v[tok, pl.ds(a + c * L, L)]
                        tval = tail_v[tr, pl.ds(th + c * L, L)]
                        out_v[orow, pl.ds(oh + c * L, L)] = (
                            jnp.where(istail, tval, val)
                            + pos_v[pr2, pl.ds(ph + c * L, L)]
                        )
                return 0

            lax.fori_loop(0, CH // L, grp_body, 0)
            off2 = pl.multiple_of((base + i * CH) // 2, 8)
            pltpu.sync_copy(out_v, out_hbm.at[pl.ds(off2, CH // 2)])

        stage(0, 0)

        def pair_body(g, _):
            i0 = 2 * g
            stage(i0 + 1, 1)
            consume(i0, 0)
            stage(i0 + 2, 0)
            consume(i0 + 1, 1)
            return 0

        lax.fori_loop(0, NCHUNK // 2 - 1, pair_body, 0)
        stage(NCHUNK - 1, 1)
        consume(NCHUNK - 2, 0)
        consume(NCHUNK - 1, 1)

    return k2(xf, tblR, pos2, tail)


def kernel(x, token_table, pos_table):
    xf = x.reshape(N).astype(jnp.int32)
    tblT = token_table.T
    tail = token_table[TAIL0:, :].reshape((V - TAIL0) // 2, 2 * E)
    pos2 = pos_table.reshape(T // 2, 2 * E)
    tblR = _sc_transpose(tblT)
    out2 = _sc_gather(xf, tblR, pos2, tail)
    return out2.reshape(B, T, E)


# speedup vs baseline: 1.0287x; 1.0287x over previous
import functools

import jax
import jax.numpy as jnp
from jax import lax
from jax.experimental import pallas as pl
from jax.experimental.pallas import tpu as pltpu
from jax.experimental.pallas import tpu_sc as plsc

B = 1024
T = 200
E = 64
N = B * T
V = 1000000

NC = 2
NS = 16
L = 16
NW = NC * NS

TK = 128
NTCH = V // TK
J1 = ((NTCH + NW - 1) // NW + 1) // 2 * 2
TAIL0 = NTCH * TK

PER_W = N // NW
CH = 320
NCHUNK = PER_W // CH

_MESH = dict(core_axis_name="c", subcore_axis_name="s",
             num_cores=NC, num_subcores=NS)


def _sc_transpose(tblT):
    @functools.partial(
        pl.kernel,
        out_type=jax.ShapeDtypeStruct((V // 2, 2 * E), jnp.float32),
        mesh=plsc.VectorSubcoreMesh(**_MESH),
        compiler_params=pltpu.CompilerParams(needs_layout_passes=False),
        scratch_types=[
            pltpu.VMEM((E, TK), jnp.float32),
            pltpu.VMEM((E, TK), jnp.float32),
            pltpu.VMEM((TK // 2, 2 * E), jnp.float32),
            pltpu.VMEM((TK // 2, 2 * E), jnp.float32),
            pltpu.SemaphoreType.DMA,
            pltpu.SemaphoreType.DMA,
        ],
    )
    def k1(tblT_hbm, tblR_hbm, sT0, sT1, oT0, oT1, ssem0, ssem1):
        sT_b = (sT0, sT1)
        oT_b = (oT0, oT1)
        ssem_b = (ssem0, ssem1)
        wid = lax.axis_index("c") * NS + lax.axis_index("s")
        ri = [lax.iota(jnp.int32, L) + c * L for c in range(E // L)]

        def chunk_of(j):
            g = wid + j * NW
            return jnp.where(g < NTCH, g, NTCH - 1)

        def stage(j, b):
            tok0 = pl.multiple_of(chunk_of(j) * TK, 128)
            pltpu.async_copy(
                tblT_hbm.at[:, pl.ds(tok0, TK)], sT_b[b], ssem_b[b]
            )

        def consume(j, b):
            sT = sT_b[b]
            oT = oT_b[b]
            tok0 = pl.multiple_of(chunk_of(j) * TK, 128)
            pltpu.make_async_copy(
                tblT_hbm.at[:, pl.ds(tok0, TK)], sT, ssem_b[b]
            ).wait()
            for t in range(TK):
                ci = jnp.full((L,), t, jnp.int32)
                for c in range(E // L):
                    oT[t // 2, pl.ds((t % 2) * E + c * L, L)] = (
                        plsc.load_gather(sT, [ri[c], ci])
                    )
            row0 = pl.multiple_of(chunk_of(j) * (TK // 2), 8)
            pltpu.sync_copy(oT, tblR_hbm.at[pl.ds(row0, TK // 2)])

        stage(0, 0)

        def pair_body(p, _):
            j0 = 2 * p
            stage(j0 + 1, 1)
            consume(j0, 0)
            stage(j0 + 2, 0)
            consume(j0 + 1, 1)
            return 0

        lax.fori_loop(0, J1 // 2 - 1, pair_body, 0)
        stage(J1 - 1, 1)
        consume(J1 - 2, 0)
        consume(J1 - 1, 1)

    return k1(tblT)


def _sc_gather(xf, tblR, pos2, tail):
    @functools.partial(
        pl.kernel,
        out_type=jax.ShapeDtypeStruct((N // 2, 2 * E), jnp.float32),
        mesh=plsc.VectorSubcoreMesh(**_MESH),
        scratch_types=[
            pltpu.VMEM((CH,), jnp.int32),
            pltpu.VMEM((CH,), jnp.int32),
            pltpu.VMEM((CH,), jnp.int32),
            pltpu.VMEM((CH,), jnp.int32),
            pltpu.VMEM((CH, 2 * E), jnp.float32),
            pltpu.VMEM((CH, 2 * E), jnp.float32),
            pltpu.VMEM((CH // 2, 2 * E), jnp.float32),
            pltpu.VMEM((T // 2, 2 * E), jnp.float32),
            pltpu.VMEM(((V - TAIL0) // 2, 2 * E), jnp.float32),
            pltpu.SemaphoreType.DMA,
            pltpu.SemaphoreType.DMA,
        ],
    )
    def k2(x_hbm, tbl_hbm, pos_hbm, tail_hbm, out_hbm,
           idx_v0, idx_v1, pidx_v0, pidx_v1, rows_v0, rows_v1,
           out_v, pos_v, tail_v, gsem0, gsem1):
        idx_b = (idx_v0, idx_v1)
        pidx_b = (pidx_v0, pidx_v1)
        rows_b = (rows_v0, rows_v1)
        gsem_b = (gsem0, gsem1)
        wid = lax.axis_index("c") * NS + lax.axis_index("s")
        base = wid * PER_W
        pltpu.sync_copy(pos_hbm, pos_v)
        pltpu.sync_copy(tail_hbm, tail_v)

        def stage(i, b):
            off = pl.multiple_of(base + i * CH, 8)
            idx_v = idx_b[b]
            pidx_v = pidx_b[b]
            pltpu.sync_copy(x_hbm.at[pl.ds(off, CH)], idx_v)

            def shift_body(v, _):
                pidx_v[pl.ds(v * L, L)] = jnp.minimum(
                    lax.shift_right_logical(idx_v[pl.ds(v * L, L)], 1),
                    TAIL0 // 2 - 1,
                )
                return 0

            lax.fori_loop(0, CH // L, shift_body, 0)
            pltpu.async_copy(tbl_hbm.at[pidx_v], rows_b[b], gsem_b[b])

        def consume(i, b):
            idx_v = idx_b[b]
            rows_v = rows_b[b]
            pltpu.make_async_copy(
                tbl_hbm.at[pidx_b[b]], rows_v, gsem_b[b]
            ).wait()
            t0 = lax.rem(i * CH, T)

            def grp_body(m, _):
                iv = idx_v[pl.ds(m * L, L)]
                for r in range(L):
                    tok = m * L + r
                    a = (iv[r] & 1) * E
                    istail = iv[r] >= TAIL0
                    toff = jnp.maximum(iv[r] - TAIL0, 0)
                    tr = lax.shift_right_logical(toff, 1)
                    th = (toff & 1) * E
                    tt = t0 + tok
                    tt = jnp.where(tt >= T, tt - T, tt)
                    tt = jnp.where(tt >= T, tt - T, tt)
                    pr2 = lax.shift_right_logical(tt, 1)
                    ph = (tt & 1) * E
                    orow = lax.shift_right_logical(tok, 1)
                    oh = (tok & 1) * E
                    for c in range(E // L):
                        val = rows_v[tok, pl.ds(a + c * L, L)]
                        tval = tail_v[tr, pl.ds(th + c * L, L)]
                        out_v[orow, pl.ds(oh + c * L, L)] = (
                            jnp.where(istail, tval, val)
                            + pos_v[pr2, pl.ds(ph + c * L, L)]
                        )
                return 0

            lax.fori_loop(0, CH // L, grp_body, 0)
            off2 = pl.multiple_of((base + i * CH) // 2, 8)
            pltpu.sync_copy(out_v, out_hbm.at[pl.ds(off2, CH // 2)])

        stage(0, 0)

        def pair_body(g, _):
            i0 = 2 * g
            stage(i0 + 1, 1)
            consume(i0, 0)
            stage(i0 + 2, 0)
            consume(i0 + 1, 1)
            return 0

        lax.fori_loop(0, NCHUNK // 2 - 1, pair_body, 0)
        stage(NCHUNK - 1, 1)
        consume(NCHUNK - 2, 0)
        consume(NCHUNK - 1, 1)

    return k2(xf, tblR, pos2, tail)


def kernel(x, token_table, pos_table):
    xf = x.reshape(N).astype(jnp.int32)
    tblT = token_table.T
    tail = token_table[TAIL0:, :].reshape((V - TAIL0) // 2, 2 * E)
    pos2 = pos_table.reshape(T // 2, 2 * E)
    tblR = _sc_transpose(tblT)
    out2 = _sc_gather(xf, tblR, pos2, tail)
    return out2.reshape(B, T, E)
